# baseline (speedup 1.0000x reference)
import functools

import jax
import jax.numpy as jnp
from jax import lax
from jax.experimental import pallas as pl
from jax.experimental.pallas import tpu as pltpu
from jax.experimental.pallas import tpu_sc as plsc

N_NODES = 10000
N_EDGES = 320000
D_FEAT = 128
D_EDGE = 16
D_OUT = 16

_NT = N_EDGES // 128
_BLK = 16000
_BT = _BLK // 128

_NC, _NS = 2, 16
_NW = _NC * _NS
_CT = 10
_CH = _CT * 128
_NCHUNK = N_EDGES // _CH
_NSLOT = -(-_NCHUNK // _NW)


def _proj_body(v_ref, w1_ref, w2_ref, p8_ref, q8_ref):
    v = v_ref[...]
    p = jnp.dot(v, w1_ref[...], preferred_element_type=jnp.float32)
    q = jnp.dot(v, w2_ref[...], preferred_element_type=jnp.float32)
    p8_ref[...] = jnp.concatenate([p] * 8, axis=1)
    q8_ref[...] = jnp.concatenate([q] * 8, axis=1)


def _base_body(eT_ref, weT_ref, bT_ref, o_ref):
    m = jnp.dot(weT_ref[...], eT_ref[...],
                preferred_element_type=jnp.float32) + bT_ref[...]
    x = m.reshape(2, 8, _BT, 128)
    o_ref[...] = x.transpose(0, 2, 1, 3)


def _sc_body(sd_hbm, p_hbm, q_hbm, base_hbm, out_hbm,
             idx_s, idx_d, rows_s, rows_d, acc, p_sp, q_sp,
             sem_g0, sem_g1, sem_o):
    wid = lax.axis_index("s") * _NC + lax.axis_index("c")
    sid = lax.axis_index("s")

    @pl.when(sid == 0)
    def _():
        pltpu.sync_copy(p_hbm.at[:, pl.ds(0, D_OUT)], p_sp)

    @pl.when(sid == 1)
    def _():
        pltpu.sync_copy(q_hbm.at[:, pl.ds(0, D_OUT)], q_sp)

    plsc.subcore_barrier()
    ii = lax.iota(jnp.int32, 16)
    c1 = (lax.shift_right_logical(ii, 3) * (_CT * 1024)
          + lax.bitwise_and(ii, 7) * 128)
    half = _NT * 1024
    sem_g = (sem_g0, sem_g1)
    cw = _CT * 1024

    def issue_gathers(chunk, buf):
        off = chunk * _CH
        pltpu.sync_copy(sd_hbm.at[0, pl.ds(off, _CH)], idx_s.at[buf])
        pltpu.sync_copy(sd_hbm.at[1, pl.ds(off, _CH)], idx_d.at[buf])
        pltpu.async_copy(p_sp.at[idx_s.at[buf]], rows_s.at[buf], sem_g[buf])
        pltpu.async_copy(q_sp.at[idx_d.at[buf]], rows_d.at[buf], sem_g[buf])

    def drain_gathers(buf):
        pltpu.make_async_copy(p_hbm.at[pl.ds(0, _CH)],
                              rows_s.at[buf], sem_g[buf]).wait()
        pltpu.make_async_copy(q_hbm.at[pl.ds(0, _CH)],
                              rows_d.at[buf], sem_g[buf]).wait()

    def drain_out():
        pltpu.make_async_copy(base_hbm.at[pl.ds(0, cw)],
                              acc.at[pl.ds(0, cw)], sem_o).wait()
        pltpu.make_async_copy(base_hbm.at[pl.ds(0, cw)],
                              acc.at[pl.ds(cw, cw)], sem_o).wait()

    def do_half(k, buf):
        chunk = wid + k * _NW

        @pl.when(chunk < _NCHUNK)
        def _():
            nxt = chunk + _NW

            @pl.when(nxt < _NCHUNK)
            def _():
                issue_gathers(nxt, buf ^ 1)

            @pl.when(k >= 1)
            def _():
                drain_out()

            t0 = chunk * _CT
            pltpu.sync_copy(base_hbm.at[pl.ds(t0 * 1024, cw)],
                            acc.at[pl.ds(0, cw)])
            pltpu.sync_copy(base_hbm.at[pl.ds(half + t0 * 1024, cw)],
                            acc.at[pl.ds(cw, cw)])
            drain_gathers(buf)

            bufv = jnp.full((16,), buf, jnp.int32)
            for t in range(_CT):
                ct1 = c1 + t * 1024

                @plsc.parallel_loop(0, 128, 1, unroll=8)
                def add_row(r0):
                    rv = lax.bitwise_and(jnp.broadcast_to(r0, (16,)) + ii,
                                         127)
                    eidx = rv + (t * 128)
                    xs = plsc.load_gather(rows_s, [bufv, eidx, ii])
                    xd = plsc.load_gather(rows_d, [bufv, eidx, ii])
                    plsc.addupdate_scatter(acc, [ct1 + rv], xs + xd)

            pltpu.async_copy(acc.at[pl.ds(0, cw)],
                             out_hbm.at[pl.ds(t0 * 1024, cw)], sem_o)
            pltpu.async_copy(acc.at[pl.ds(cw, cw)],
                             out_hbm.at[pl.ds(half + t0 * 1024, cw)], sem_o)

    issue_gathers(wid, 0)

    def pair(j, carry):
        do_half(2 * j, 0)
        do_half(2 * j + 1, 1)
        return carry

    lax.fori_loop(0, _NSLOT // 2, pair, 0)
    drain_out()


def kernel(e, v, edges, W, b):
    W1 = W[:D_FEAT]
    W2 = W[D_FEAT:2 * D_FEAT]
    WeT = W[2 * D_FEAT:].T
    bT = b.reshape(D_OUT, 1)
    eT = e.T
    sd = edges.T

    P8, Q8 = pl.pallas_call(
        _proj_body,
        out_shape=[jax.ShapeDtypeStruct((N_NODES, D_FEAT), jnp.float32)] * 2,
    )(v, W1, W2)

    base4 = pl.pallas_call(
        _base_body,
        grid=(N_EDGES // _BLK,),
        in_specs=[
            pl.BlockSpec((D_OUT, _BLK), lambda i: (0, i)),
            pl.BlockSpec((D_OUT, D_OUT), lambda i: (0, 0)),
            pl.BlockSpec((D_OUT, 1), lambda i: (0, 0)),
        ],
        out_specs=pl.BlockSpec((2, _BT, 8, 128), lambda i: (0, i, 0, 0)),
        out_shape=jax.ShapeDtypeStruct((2, _NT, 8, 128), jnp.float32),
    )(eT, WeT, bT)

    mesh = plsc.VectorSubcoreMesh(
        core_axis_name="c", subcore_axis_name="s",
        num_cores=_NC, num_subcores=_NS)
    sc = pl.kernel(
        _sc_body,
        out_type=jax.ShapeDtypeStruct((2 * _NT * 8 * 128,), jnp.float32),
        mesh=mesh,
        scratch_types=[
            pltpu.VMEM((2, _CH), jnp.int32),
            pltpu.VMEM((2, _CH), jnp.int32),
            pltpu.VMEM((2, _CH, D_OUT), jnp.float32),
            pltpu.VMEM((2, _CH, D_OUT), jnp.float32),
            pltpu.VMEM((2 * _CT * 8 * 128,), jnp.float32),
            pltpu.VMEM_SHARED((N_NODES, D_OUT), jnp.float32),
            pltpu.VMEM_SHARED((N_NODES, D_OUT), jnp.float32),
            pltpu.SemaphoreType.DMA,
            pltpu.SemaphoreType.DMA,
            pltpu.SemaphoreType.DMA,
        ],
        compiler_params=pltpu.CompilerParams(
            use_tc_tiling_on_sc=False, needs_layout_passes=False),
    )
    out_flat = sc(sd, P8, Q8, base4.reshape(-1))
    return (out_flat.reshape(2, _NT, 8, 128)
            .transpose(1, 3, 0, 2).reshape(N_EDGES, D_OUT))


# speedup vs baseline: 1.1355x; 1.1355x over previous
import functools

import jax
import jax.numpy as jnp
from jax import lax
from jax.experimental import pallas as pl
from jax.experimental.pallas import tpu as pltpu
from jax.experimental.pallas import tpu_sc as plsc

N_NODES = 10000
N_EDGES = 320000
D_FEAT = 128
D_EDGE = 16
D_OUT = 16

_NT = N_EDGES // 128
_BLK = 16000
_BT = _BLK // 128

_NC, _NS = 2, 16
_NW = _NC * _NS
_CT = 10
_CH = _CT * 128
_NCHUNK = N_EDGES // _CH
_NSLOT = -(-_NCHUNK // _NW)


def _proj_body(v_ref, w1_ref, w2_ref, p_ref, q_ref):
    v = v_ref[...]
    p_ref[...] = jnp.dot(v, w1_ref[...], preferred_element_type=jnp.float32)
    q_ref[...] = jnp.dot(v, w2_ref[...], preferred_element_type=jnp.float32)


def _base_body(eT_ref, weT_ref, bT_ref, o_ref):
    m = jnp.dot(weT_ref[...], eT_ref[...],
                preferred_element_type=jnp.float32) + bT_ref[...]
    x = m.reshape(2, 8, _BT, 128)
    o_ref[...] = x.transpose(0, 2, 1, 3)


def _sc_body(sd_hbm, p_hbm, q_hbm, base_hbm, out_hbm,
             idx_s, idx_d, rows_s, rows_d, acc, p_sp, q_sp,
             sem_g0, sem_g1, sem_o):
    wid = lax.axis_index("s") * _NC + lax.axis_index("c")
    sid = lax.axis_index("s")

    @pl.when(sid == 0)
    def _():
        pltpu.sync_copy(p_hbm, p_sp)

    @pl.when(sid == 1)
    def _():
        pltpu.sync_copy(q_hbm, q_sp)

    plsc.subcore_barrier()
    ii = lax.iota(jnp.int32, 16)
    c1 = (lax.shift_right_logical(ii, 3) * (_CT * 1024)
          + lax.bitwise_and(ii, 7) * 128)
    half = _NT * 1024
    sem_g = (sem_g0, sem_g1)
    cw = _CT * 1024

    def issue_gathers(chunk, buf):
        off = chunk * _CH
        pltpu.sync_copy(sd_hbm.at[0, pl.ds(off, _CH)], idx_s.at[buf])
        pltpu.sync_copy(sd_hbm.at[1, pl.ds(off, _CH)], idx_d.at[buf])
        pltpu.async_copy(p_sp.at[idx_s.at[buf]], rows_s.at[buf], sem_g[buf])
        pltpu.async_copy(q_sp.at[idx_d.at[buf]], rows_d.at[buf], sem_g[buf])

    def drain_gathers(buf):
        pltpu.make_async_copy(p_hbm.at[pl.ds(0, _CH)],
                              rows_s.at[buf], sem_g[buf]).wait()
        pltpu.make_async_copy(q_hbm.at[pl.ds(0, _CH)],
                              rows_d.at[buf], sem_g[buf]).wait()

    def drain_out():
        pltpu.make_async_copy(base_hbm.at[pl.ds(0, cw)],
                              acc.at[pl.ds(0, cw)], sem_o).wait()
        pltpu.make_async_copy(base_hbm.at[pl.ds(0, cw)],
                              acc.at[pl.ds(cw, cw)], sem_o).wait()

    def do_half(k, buf):
        chunk = wid + k * _NW

        @pl.when(chunk < _NCHUNK)
        def _():
            nxt = chunk + _NW

            @pl.when(nxt < _NCHUNK)
            def _():
                issue_gathers(nxt, buf ^ 1)

            @pl.when(k >= 1)
            def _():
                drain_out()

            t0 = chunk * _CT
            pltpu.sync_copy(base_hbm.at[pl.ds(t0 * 1024, cw)],
                            acc.at[pl.ds(0, cw)])
            pltpu.sync_copy(base_hbm.at[pl.ds(half + t0 * 1024, cw)],
                            acc.at[pl.ds(cw, cw)])
            drain_gathers(buf)

            bufv = jnp.full((16,), buf, jnp.int32)
            for t in range(_CT):
                ct1 = c1 + t * 1024

                @plsc.parallel_loop(0, 128, 1, unroll=8)
                def add_row(r0):
                    rv = lax.bitwise_and(jnp.broadcast_to(r0, (16,)) + ii,
                                         127)
                    eidx = rv + (t * 128)
                    xs = plsc.load_gather(rows_s, [bufv, eidx, ii])
                    xd = plsc.load_gather(rows_d, [bufv, eidx, ii])
                    plsc.addupdate_scatter(acc, [ct1 + rv], xs + xd)

            pltpu.async_copy(acc.at[pl.ds(0, cw)],
                             out_hbm.at[pl.ds(t0 * 1024, cw)], sem_o)
            pltpu.async_copy(acc.at[pl.ds(cw, cw)],
                             out_hbm.at[pl.ds(half + t0 * 1024, cw)], sem_o)

    issue_gathers(wid, 0)

    def pair(j, carry):
        do_half(2 * j, 0)
        do_half(2 * j + 1, 1)
        return carry

    lax.fori_loop(0, _NSLOT // 2, pair, 0)
    drain_out()


def kernel(e, v, edges, W, b):
    W1 = W[:D_FEAT]
    W2 = W[D_FEAT:2 * D_FEAT]
    WeT = W[2 * D_FEAT:].T
    bT = b.reshape(D_OUT, 1)
    eT = e.T
    sd = edges.T

    P, Q = pl.pallas_call(
        _proj_body,
        out_shape=[jax.ShapeDtypeStruct((N_NODES, D_OUT), jnp.float32)] * 2,
    )(v, W1, W2)

    base4 = pl.pallas_call(
        _base_body,
        grid=(N_EDGES // _BLK,),
        in_specs=[
            pl.BlockSpec((D_OUT, _BLK), lambda i: (0, i)),
            pl.BlockSpec((D_OUT, D_OUT), lambda i: (0, 0)),
            pl.BlockSpec((D_OUT, 1), lambda i: (0, 0)),
        ],
        out_specs=pl.BlockSpec((2, _BT, 8, 128), lambda i: (0, i, 0, 0)),
        out_shape=jax.ShapeDtypeStruct((2, _NT, 8, 128), jnp.float32),
    )(eT, WeT, bT)

    mesh = plsc.VectorSubcoreMesh(
        core_axis_name="c", subcore_axis_name="s",
        num_cores=_NC, num_subcores=_NS)
    sc = pl.kernel(
        _sc_body,
        out_type=jax.ShapeDtypeStruct((2 * _NT * 8 * 128,), jnp.float32),
        mesh=mesh,
        scratch_types=[
            pltpu.VMEM((2, _CH), jnp.int32),
            pltpu.VMEM((2, _CH), jnp.int32),
            pltpu.VMEM((2, _CH, D_OUT), jnp.float32),
            pltpu.VMEM((2, _CH, D_OUT), jnp.float32),
            pltpu.VMEM((2 * _CT * 8 * 128,), jnp.float32),
            pltpu.VMEM_SHARED((N_NODES, D_OUT), jnp.float32),
            pltpu.VMEM_SHARED((N_NODES, D_OUT), jnp.float32),
            pltpu.SemaphoreType.DMA,
            pltpu.SemaphoreType.DMA,
            pltpu.SemaphoreType.DMA,
        ],
        compiler_params=pltpu.CompilerParams(
            use_tc_tiling_on_sc=False, needs_layout_passes=False),
    )
    out_flat = sc(sd, P, Q, base4.reshape(-1))
    return (out_flat.reshape(2, _NT, 8, 128)
            .transpose(1, 3, 0, 2).reshape(N_EDGES, D_OUT))
